# baseline (speedup 1.0000x reference)
import functools

import jax
import jax.numpy as jnp
from jax import lax
from jax.experimental import pallas as pl
from jax.experimental.pallas import tpu as pltpu
from jax.experimental.pallas import tpu_sc as plsc

NC = 2
NS = 16
L = 16
NW = NC * NS
B = 16384
D = 32
SR = 128
RPS = SR // D
BPW = B // NW
CHUNK = 128
NCHUNK = BPW // CHUNK
GPC = CHUNK // L

_LOG1P = (
    3.910905549409094e-08, 0.9999936302585134, -0.4998254986434647,
    0.33144665224336606, -0.2394333707458602, 0.16499812983396112,
    -0.09229041738050231, 0.03426459995555095, -0.006006605050865348,
)


def _log1p_poly(t):
    acc = jnp.full_like(t, _LOG1P[-1])
    for c in reversed(_LOG1P[:-1]):
        acc = acc * t + jnp.float32(c)
    return acc


@functools.cache
def _build_bpr_sc():
  mesh = plsc.VectorSubcoreMesh(
      core_axis_name="c", subcore_axis_name="s", num_cores=NC, num_subcores=NS)

  @functools.partial(
      pl.kernel,
      out_type=jax.ShapeDtypeStruct((NW, L), jnp.float32),
      mesh=mesh,
      scratch_types=[
          pltpu.VMEM((NCHUNK, CHUNK), jnp.int32),
          pltpu.VMEM((NCHUNK, CHUNK), jnp.int32),
          pltpu.VMEM((NCHUNK, CHUNK), jnp.int32),
          pltpu.VMEM((BPW,), jnp.int32),
          pltpu.VMEM((BPW,), jnp.int32),
          pltpu.VMEM((BPW,), jnp.int32),
          pltpu.VMEM((CHUNK, SR), jnp.float32),
          pltpu.VMEM((CHUNK, SR), jnp.float32),
          pltpu.VMEM((CHUNK, SR), jnp.float32),
          pltpu.VMEM((L,), jnp.float32),
          pltpu.SemaphoreType.DMA,
      ],
      compiler_params=pltpu.CompilerParams(needs_layout_passes=False),
  )
  def _bpr_sc(us_hbm, is_hbm, js_hbm, uo_hbm, io_hbm, jo_hbm,
              w_hbm, h_hbm, out_hbm,
              us_v, is_v, js_v, uo_v, io_v, jo_v,
              ue_v, ie_v, je_v, o_v, sem):
    wid = lax.axis_index("s") * NC + lax.axis_index("c")

    pltpu.sync_copy(us_hbm.at[wid], us_v)
    pltpu.sync_copy(is_hbm.at[wid], is_v)
    pltpu.sync_copy(js_hbm.at[wid], js_v)
    pltpu.sync_copy(uo_hbm.at[wid], uo_v)
    pltpu.sync_copy(io_hbm.at[wid], io_v)
    pltpu.sync_copy(jo_hbm.at[wid], jo_v)

    iota = lax.iota(jnp.int32, L)
    zero = jnp.zeros((L,), jnp.float32)
    acc = zero

    for c in range(NCHUNK):
      cps = [pltpu.async_copy(w_hbm.at[us_v.at[c]], ue_v, sem),
             pltpu.async_copy(h_hbm.at[is_v.at[c]], ie_v, sem),
             pltpu.async_copy(h_hbm.at[js_v.at[c]], je_v, sem)]
      for cp in cps:
        cp.wait()

      def body(g, a, c=c):
        r_idx = g * L + iota
        cu = uo_v[pl.ds(c * CHUNK + g * L, L)] * D
        ci = io_v[pl.ds(c * CHUNK + g * L, L)] * D
        cj = jo_v[pl.ds(c * CHUNK + g * L, L)] * D
        x = zero
        for d in range(D):
          ue = plsc.load_gather(ue_v, [r_idx, cu + d])
          ie = plsc.load_gather(ie_v, [r_idx, ci + d])
          je = plsc.load_gather(je_v, [r_idx, cj + d])
          x = x + ue * (ie - je)
        t = jnp.exp(-jnp.abs(x))
        return a + jnp.minimum(x, 0.0) - _log1p_poly(t)

      acc = lax.fori_loop(0, GPC, body, acc)

    o_v[...] = acc
    pltpu.sync_copy(o_v, out_hbm.at[wid])

  return _bpr_sc


def kernel(u, i, j, W, H):
    u = u.astype(jnp.int32)
    i = i.astype(jnp.int32)
    j = j.astype(jnp.int32)
    us = (u // RPS).reshape(NW, NCHUNK, CHUNK)
    is_ = (i // RPS).reshape(NW, NCHUNK, CHUNK)
    js = (j // RPS).reshape(NW, NCHUNK, CHUNK)
    uo = (u % RPS).reshape(NW, BPW)
    io = (i % RPS).reshape(NW, BPW)
    jo = (j % RPS).reshape(NW, BPW)
    w4 = W.reshape(-1, SR)
    h4 = H.reshape(-1, SR)
    partials = _build_bpr_sc()(us, is_, js, uo, io, jo, w4, h4)
    return -jnp.sum(partials)


# speedup vs baseline: 1.4739x; 1.4739x over previous
import functools

import jax
import jax.numpy as jnp
from jax import lax
from jax.experimental import pallas as pl
from jax.experimental.pallas import tpu as pltpu
from jax.experimental.pallas import tpu_sc as plsc

NC = 2
NS = 16
L = 16
NW = NC * NS
B = 16384
D = 32
BPW = B // NW
CHUNK = 128
NCHUNK = BPW // CHUNK
GPC = CHUNK // L

_LOG1P = (
    3.910905549409094e-08, 0.9999936302585134, -0.4998254986434647,
    0.33144665224336606, -0.2394333707458602, 0.16499812983396112,
    -0.09229041738050231, 0.03426459995555095, -0.006006605050865348,
)


def _log1p_poly(t):
    acc = jnp.full_like(t, _LOG1P[-1])
    for c in reversed(_LOG1P[:-1]):
        acc = acc * t + jnp.float32(c)
    return acc


@functools.cache
def _build_bpr_sc():
  mesh = plsc.VectorSubcoreMesh(
      core_axis_name="c", subcore_axis_name="s", num_cores=NC, num_subcores=NS)

  @functools.partial(
      pl.kernel,
      out_type=jax.ShapeDtypeStruct((NW, L), jnp.float32),
      mesh=mesh,
      scratch_types=[
          pltpu.VMEM((BPW,), jnp.int32),
          pltpu.VMEM((BPW,), jnp.int32),
          pltpu.VMEM((BPW,), jnp.int32),
          pltpu.VMEM((CHUNK, D), jnp.float32),
          pltpu.VMEM((CHUNK, D), jnp.float32),
          pltpu.VMEM((CHUNK, D), jnp.float32),
          pltpu.VMEM((L,), jnp.float32),
          pltpu.SemaphoreType.DMA,
      ],
      compiler_params=pltpu.CompilerParams(needs_layout_passes=False),
  )
  def _bpr_sc(u_hbm, i_hbm, j_hbm, w_hbm, h_hbm, out_hbm,
              u_v, i_v, j_v,
              ue_v, ie_v, je_v, o_v, sem):
    wid = lax.axis_index("s") * NC + lax.axis_index("c")

    pltpu.sync_copy(u_hbm.at[wid], u_v)
    pltpu.sync_copy(i_hbm.at[wid], i_v)
    pltpu.sync_copy(j_hbm.at[wid], j_v)

    iota = lax.iota(jnp.int32, L)
    zero = jnp.zeros((L,), jnp.float32)
    acc = zero

    for c in range(NCHUNK):
      base = c * CHUNK

      def issue(g, carry, base=base):
        uvec = u_v[pl.ds(base + g * L, L)]
        ivec = i_v[pl.ds(base + g * L, L)]
        jvec = j_v[pl.ds(base + g * L, L)]
        for l in range(L):
          r = g * L + l
          pltpu.async_copy(w_hbm.at[uvec[l]], ue_v.at[r], sem)
          pltpu.async_copy(h_hbm.at[ivec[l]], ie_v.at[r], sem)
          pltpu.async_copy(h_hbm.at[jvec[l]], je_v.at[r], sem)
        return carry

      lax.fori_loop(0, GPC, issue, 0)
      pltpu.make_async_copy(w_hbm.at[pl.ds(0, CHUNK)], ue_v, sem).wait()
      pltpu.make_async_copy(w_hbm.at[pl.ds(0, CHUNK)], ie_v, sem).wait()
      pltpu.make_async_copy(w_hbm.at[pl.ds(0, CHUNK)], je_v, sem).wait()

      def body(g, a):
        r_idx = g * L + iota
        x = zero
        for d in range(D):
          d_idx = jnp.full((L,), d, jnp.int32)
          ue = plsc.load_gather(ue_v, [r_idx, d_idx])
          ie = plsc.load_gather(ie_v, [r_idx, d_idx])
          je = plsc.load_gather(je_v, [r_idx, d_idx])
          x = x + ue * (ie - je)
        t = jnp.exp(-jnp.abs(x))
        return a + jnp.minimum(x, 0.0) - _log1p_poly(t)

      acc = lax.fori_loop(0, GPC, body, acc)

    o_v[...] = acc
    pltpu.sync_copy(o_v, out_hbm.at[wid])

  return _bpr_sc


def kernel(u, i, j, W, H):
    u2 = u.astype(jnp.int32).reshape(NW, BPW)
    i2 = i.astype(jnp.int32).reshape(NW, BPW)
    j2 = j.astype(jnp.int32).reshape(NW, BPW)
    partials = _build_bpr_sc()(u2, i2, j2, W, H)
    return -jnp.sum(partials)


# speedup vs baseline: 1.4874x; 1.0091x over previous
import functools

import jax
import jax.numpy as jnp
from jax import lax
from jax.experimental import pallas as pl
from jax.experimental.pallas import tpu as pltpu
from jax.experimental.pallas import tpu_sc as plsc

NC = 2
NS = 16
L = 16
NW = NC * NS
B = 16384
D = 32
BPW = B // NW
CHUNK = 128
NCHUNK = BPW // CHUNK
GPC = CHUNK // L

_LOG1P = (
    3.910905549409094e-08, 0.9999936302585134, -0.4998254986434647,
    0.33144665224336606, -0.2394333707458602, 0.16499812983396112,
    -0.09229041738050231, 0.03426459995555095, -0.006006605050865348,
)


def _log1p_poly(t):
    acc = jnp.full_like(t, _LOG1P[-1])
    for c in reversed(_LOG1P[:-1]):
        acc = acc * t + jnp.float32(c)
    return acc


@functools.cache
def _build_bpr_sc():
  mesh = plsc.VectorSubcoreMesh(
      core_axis_name="c", subcore_axis_name="s", num_cores=NC, num_subcores=NS)

  @functools.partial(
      pl.kernel,
      out_type=jax.ShapeDtypeStruct((NW, L), jnp.float32),
      mesh=mesh,
      scratch_types=[
          pltpu.VMEM((BPW,), jnp.int32),
          pltpu.VMEM((BPW,), jnp.int32),
          pltpu.VMEM((BPW,), jnp.int32),
          [pltpu.VMEM((CHUNK, D), jnp.float32)] * 6,
          pltpu.VMEM((L,), jnp.float32),
          [pltpu.SemaphoreType.DMA] * 6,
      ],
      compiler_params=pltpu.CompilerParams(needs_layout_passes=False),
  )
  def _bpr_sc(u_hbm, i_hbm, j_hbm, w_hbm, h_hbm, out_hbm,
              u_v, i_v, j_v, bufs, o_v, sems):
    wid = lax.axis_index("s") * NC + lax.axis_index("c")

    pltpu.sync_copy(u_hbm.at[wid], u_v)
    pltpu.sync_copy(i_hbm.at[wid], i_v)
    pltpu.sync_copy(j_hbm.at[wid], j_v)

    iota = lax.iota(jnp.int32, L)
    zero = jnp.zeros((L,), jnp.float32)

    def issue_chunk(c, par):
      ue_v, ie_v, je_v = bufs[par * 3:par * 3 + 3]
      su, si, sj = sems[par * 3:par * 3 + 3]

      def issue(g, carry):
        base = c * CHUNK + g * L
        uvec = u_v[pl.ds(base, L)]
        ivec = i_v[pl.ds(base, L)]
        jvec = j_v[pl.ds(base, L)]
        for l in range(L):
          r = g * L + l
          pltpu.async_copy(w_hbm.at[uvec[l]], ue_v.at[r], su)
          pltpu.async_copy(h_hbm.at[ivec[l]], ie_v.at[r], si)
          pltpu.async_copy(h_hbm.at[jvec[l]], je_v.at[r], sj)
        return carry

      lax.fori_loop(0, GPC, issue, 0)

    def drain_chunk(par):
      ue_v, ie_v, je_v = bufs[par * 3:par * 3 + 3]
      su, si, sj = sems[par * 3:par * 3 + 3]
      pltpu.make_async_copy(w_hbm.at[pl.ds(0, CHUNK)], ue_v, su).wait()
      pltpu.make_async_copy(w_hbm.at[pl.ds(0, CHUNK)], ie_v, si).wait()
      pltpu.make_async_copy(w_hbm.at[pl.ds(0, CHUNK)], je_v, sj).wait()

    def compute_chunk(par, acc):
      ue_v, ie_v, je_v = bufs[par * 3:par * 3 + 3]

      def body(g, a):
        r_idx = g * L + iota
        x = zero
        for d in range(D):
          d_idx = jnp.full((L,), d, jnp.int32)
          ue = plsc.load_gather(ue_v, [r_idx, d_idx])
          ie = plsc.load_gather(ie_v, [r_idx, d_idx])
          je = plsc.load_gather(je_v, [r_idx, d_idx])
          x = x + ue * (ie - je)
        t = jnp.exp(-jnp.abs(x))
        return a + jnp.minimum(x, 0.0) - _log1p_poly(t)

      return lax.fori_loop(0, GPC, body, acc)

    acc = zero
    issue_chunk(0, 0)
    for c in range(NCHUNK):
      if c + 1 < NCHUNK:
        issue_chunk(c + 1, (c + 1) % 2)
      drain_chunk(c % 2)
      acc = compute_chunk(c % 2, acc)

    o_v[...] = acc
    pltpu.sync_copy(o_v, out_hbm.at[wid])

  return _bpr_sc


def kernel(u, i, j, W, H):
    u2 = u.astype(jnp.int32).reshape(NW, BPW)
    i2 = i.astype(jnp.int32).reshape(NW, BPW)
    j2 = j.astype(jnp.int32).reshape(NW, BPW)
    partials = _build_bpr_sc()(u2, i2, j2, W, H)
    return -jnp.sum(partials)
